# Initial kernel scaffold; baseline (speedup 1.0000x reference)
import jax
import jax.numpy as jnp
from jax.experimental import pallas as pl


def kernel(x, edge_index, enc1_W, enc1_as, enc1_ad, enc1_b, enc2_W, enc2_as, enc2_ad, enc2_b, dec1_W, dec1_as, dec1_ad, dec1_b, dec2_W, dec2_as, dec2_ad, dec2_b, cluster):
    raise NotImplementedError("write your pallas kernel here")


# speedup vs baseline: 18.8340x; 18.8340x over previous
import functools

import jax
import jax.numpy as jnp
from jax import lax
from jax.experimental import pallas as pl
from jax.experimental.pallas import tpu as pltpu
from jax.experimental.pallas import tpu_sc as plsc

N = 10000
E_RAW = 320000
E_TOT = E_RAW + N
NW = 32
C = 128
CH = -(-E_TOT // (NW * C))
PER_W = CH * C
E_PAD = PER_W * NW
BLK = 1000
GRID = N // BLK
RPT = N // 16


def _sc_gat(an, ad, h, src, dst):
    Dp = h.shape[1]
    mesh = plsc.VectorSubcoreMesh(core_axis_name="c", subcore_axis_name="s")

    @functools.partial(
        pl.kernel,
        out_type=(
            jax.ShapeDtypeStruct((2, N, Dp), jnp.float32),
            jax.ShapeDtypeStruct((2, N), jnp.float32),
        ),
        mesh=mesh,
        compiler_params=pltpu.CompilerParams(needs_layout_passes=False),
        scratch_types=(
            pltpu.VMEM((N,), jnp.float32),
            pltpu.VMEM((N,), jnp.float32),
            pltpu.VMEM((C,), jnp.int32),
            pltpu.VMEM((C,), jnp.int32),
            pltpu.VMEM((C,), jnp.float32),
            pltpu.VMEM((C, Dp), jnp.float32),
            pltpu.VMEM((16, Dp), jnp.float32),
            pltpu.VMEM((2000,), jnp.float32),
            pltpu.VMEM_SHARED((N, Dp), jnp.float32),
            pltpu.VMEM_SHARED((N,), jnp.float32),
            pltpu.SemaphoreType.DMA,
        ),
    )
    def k(an_h, ad_h, h_h, src_h, dst_h, acc_o, s_o,
          an_v, ad_v, src_v, dst_v, w_v, rows_v, zbuf, zflat, acc_sh, s_sh,
          sem):
        core = lax.axis_index("c")
        sid = lax.axis_index("s")
        i32 = jnp.int32
        iota = lax.iota(i32, 16)
        zeros16 = jnp.zeros((16,), jnp.float32)

        pltpu.sync_copy(an_h, an_v)
        pltpu.sync_copy(ad_h, ad_v)

        def zf(i, _):
            zflat[pl.ds(i * 16, 16)] = zeros16
            return 0
        lax.fori_loop(0, 125, zf, 0)

        for r in range(16):
            for j in range(Dp // 16):
                zbuf[r, pl.ds(16 * j, 16)] = zeros16

        def za(kk, _):
            pltpu.sync_copy(zbuf, acc_sh.at[pl.ds(sid * 624 + kk * 16, 16)])
            return 0
        lax.fori_loop(0, 39, za, 0)

        @pl.when(sid == 15)
        def _ztail():
            pltpu.sync_copy(zbuf, acc_sh.at[pl.ds(9984, 16)])

        @pl.when(sid == 0)
        def _zs():
            for kk in range(5):
                pltpu.sync_copy(zflat, s_sh.at[pl.ds(kk * 2000, 2000)])

        plsc.subcore_barrier()

        wid = core * 16 + sid
        base_w = wid * PER_W

        def chunk(g, _):
            base = base_w + g * C
            pltpu.sync_copy(src_h.at[pl.ds(base, C)], src_v)
            pltpu.sync_copy(dst_h.at[pl.ds(base, C)], dst_v)
            pltpu.async_copy(h_h.at[src_v], rows_v, sem).wait()
            for i in range(C // 16):
                sv = src_v[pl.ds(16 * i, 16)]
                dv = dst_v[pl.ds(16 * i, 16)]
                a = plsc.load_gather(an_v, [sv])
                b2 = plsc.load_gather(ad_v, [dv])
                e = a + b2
                e = jnp.where(e >= 0.0, e, 0.2 * e)
                w = jnp.exp(e)
                eid = base + (16 * i) + iota
                w = jnp.where(eid < E_TOT, w, 0.0)
                w_v[pl.ds(16 * i, 16)] = w
            pltpu.sync_copy(w_v, s_sh.at[dst_v], add=True)

            def scale(r, _):
                wb = plsc.load_gather(w_v, [jnp.full((16,), r, i32)])
                for j in range(Dp // 16):
                    rv = rows_v[r, pl.ds(16 * j, 16)]
                    rows_v[r, pl.ds(16 * j, 16)] = rv * wb
                return 0
            lax.fori_loop(0, C, scale, 0)
            pltpu.sync_copy(rows_v, acc_sh.at[dst_v], add=True)
            return 0
        lax.fori_loop(0, CH, chunk, 0)

        plsc.subcore_barrier()
        pltpu.sync_copy(acc_sh.at[pl.ds(sid * 624, 624)],
                        acc_o.at[core, pl.ds(sid * 624, 624)])

        @pl.when(sid == 15)
        def _wtail():
            pltpu.sync_copy(acc_sh.at[pl.ds(9984, 16)],
                            acc_o.at[core, pl.ds(9984, 16)])

        @pl.when(sid == 0)
        def _ws():
            pltpu.sync_copy(s_sh, s_o.at[core])

    return k(an, ad, h, src, dst)


def _tc_dense(parts_in, s2, bvec, W, asv, adv, relu, emit_t):
    Dout, Din = W.shape
    np_out = 2 if Dout > 128 else 1
    Dp_out = Dout // np_out
    n_in = len(parts_in)
    pro = s2 is not None

    out_shape = [jax.ShapeDtypeStruct((N, Dp_out), jnp.float32)
                 for _ in range(np_out)]
    out_shape += [jax.ShapeDtypeStruct((N, 1), jnp.float32)] * 2
    if emit_t:
        out_shape.append(jax.ShapeDtypeStruct((N, Din), jnp.float32))

    in_specs = []
    ops = [ ]
    if pro:
        Dpp = Din // n_in
        for p in parts_in:
            in_specs.append(pl.BlockSpec((2, BLK, Dpp), lambda i: (0, i, 0)))
            ops.append(p)
        in_specs.append(pl.BlockSpec((BLK, 2), lambda i: (i, 0)))
        ops.append(s2)
        in_specs.append(pl.BlockSpec((1, Din), lambda i: (0, 0)))
        ops.append(bvec)
    else:
        in_specs.append(pl.BlockSpec((BLK, Din), lambda i: (i, 0)))
        ops.append(parts_in[0])
    in_specs.append(pl.BlockSpec((Dout, Din), lambda i: (0, 0)))
    ops.append(W)
    in_specs.append(pl.BlockSpec((1, Dout), lambda i: (0, 0)))
    ops.append(asv)
    in_specs.append(pl.BlockSpec((1, Dout), lambda i: (0, 0)))
    ops.append(adv)

    out_specs = [pl.BlockSpec((BLK, Dp_out), lambda i: (i, 0))
                 for _ in range(np_out)]
    out_specs += [pl.BlockSpec((BLK, 1), lambda i: (i, 0))] * 2
    if emit_t:
        out_specs.append(pl.BlockSpec((BLK, Din), lambda i: (i, 0)))

    def body(*refs):
        pos = 0
        if pro:
            prefs = refs[:n_in]
            s_ref = refs[n_in]
            b_ref = refs[n_in + 1]
            pos = n_in + 2
        else:
            prefs = (refs[0],)
            pos = 1
        W_ref, as_ref, ad_ref = refs[pos], refs[pos + 1], refs[pos + 2]
        out_refs = refs[pos + 3:]
        if pro:
            cols = [pr[0] + pr[1] for pr in prefs]
            t = cols[0] if n_in == 1 else jnp.concatenate(cols, axis=1)
            ssum = s_ref[:, 0:1] + s_ref[:, 1:2]
            t = t / ssum + b_ref[...]
            if relu:
                t = jnp.maximum(t, 0.0)
        else:
            t = prefs[0][...]
        h = lax.dot_general(t, W_ref[...], (((1,), (1,)), ((), ())),
                            preferred_element_type=jnp.float32)
        an = jnp.sum(h * as_ref[...], axis=1, keepdims=True)
        ad = jnp.sum(h * ad_ref[...], axis=1, keepdims=True)
        o = 0
        for p in range(np_out):
            out_refs[o][...] = h[:, p * Dp_out:(p + 1) * Dp_out]
            o += 1
        out_refs[o][...] = an
        o += 1
        out_refs[o][...] = ad
        o += 1
        if emit_t:
            out_refs[o][...] = t

    return pl.pallas_call(
        body, grid=(GRID,), in_specs=in_specs, out_specs=out_specs,
        out_shape=out_shape)(*ops)


def _tc_combine(acc, s2, bvec):
    Dp = acc.shape[2]

    def body(a_ref, s_ref, b_ref, o_ref):
        ssum = s_ref[:, 0:1] + s_ref[:, 1:2]
        o_ref[...] = (a_ref[0] + a_ref[1]) / ssum + b_ref[...]

    return pl.pallas_call(
        body, grid=(GRID,),
        in_specs=[pl.BlockSpec((2, BLK, Dp), lambda i: (0, i, 0)),
                  pl.BlockSpec((BLK, 2), lambda i: (i, 0)),
                  pl.BlockSpec((1, Dp), lambda i: (0, 0))],
        out_specs=pl.BlockSpec((BLK, Dp), lambda i: (i, 0)),
        out_shape=jax.ShapeDtypeStruct((N, Dp), jnp.float32),
    )(acc, s2, bvec)


def _tc_q(z, cluster):
    K, Zd = cluster.shape

    def body(z_ref, c_ref, o_ref):
        zb = z_ref[...]
        cb = c_ref[...]
        zz = jnp.sum(zb * zb, axis=1, keepdims=True)
        cc = jnp.sum(cb * cb, axis=1).reshape(1, K)
        d2 = zz + cc - 2.0 * lax.dot_general(
            zb, cb, (((1,), (1,)), ((), ())),
            preferred_element_type=jnp.float32)
        q = jnp.sqrt(1.0 / (1.0 + d2))
        o_ref[...] = q / jnp.sum(q, axis=1, keepdims=True)

    return pl.pallas_call(
        body, grid=(GRID,),
        in_specs=[pl.BlockSpec((BLK, Zd), lambda i: (i, 0)),
                  pl.BlockSpec((K, Zd), lambda i: (0, 0))],
        out_specs=pl.BlockSpec((BLK, K), lambda i: (i, 0)),
        out_shape=jax.ShapeDtypeStruct((N, K), jnp.float32),
    )(z, cluster)


def kernel(x, edge_index, enc1_W, enc1_as, enc1_ad, enc1_b,
           enc2_W, enc2_as, enc2_ad, enc2_b,
           dec1_W, dec1_as, dec1_ad, dec1_b,
           dec2_W, dec2_as, dec2_ad, dec2_b, cluster):
    ei = edge_index.astype(jnp.int32)
    loops = jnp.arange(N, dtype=jnp.int32)
    n_pad = E_PAD - E_TOT
    pad_idx = (jnp.arange(n_pad, dtype=jnp.int32) * 97) % N
    src = jnp.concatenate([ei[0], loops, pad_idx])
    dst = jnp.concatenate([ei[1], loops, pad_idx])

    def gat_layer(parts_in, s_prev, b_prev, W, a_s, a_d, relu, emit_t):
        outs = _tc_dense(parts_in, s_prev, b_prev, W,
                         a_s.reshape(1, -1), a_d.reshape(1, -1), relu, emit_t)
        np_out = 2 if W.shape[0] > 128 else 1
        hparts = outs[:np_out]
        an = outs[np_out].reshape(N)
        ad = outs[np_out + 1].reshape(N)
        t = outs[np_out + 2] if emit_t else None
        accs = []
        s_pair = None
        for p, hp in enumerate(hparts):
            d = hp.shape[1]
            if d < 128:
                hp = jnp.pad(hp, ((0, 0), (0, 128 - d)))
            acc, s_parts = _sc_gat(an, ad, hp, src, dst)
            if d < 128:
                acc = acc[:, :, :d]
            accs.append(acc)
            if p == 0:
                s_pair = s_parts.T
        return accs, s_pair, t

    a1, s1, _ = gat_layer([x], None, None, enc1_W, enc1_as, enc1_ad,
                          False, False)
    a2, s2, _ = gat_layer(a1, s1, enc1_b.reshape(1, -1), enc2_W, enc2_as,
                          enc2_ad, True, False)
    a3, s3, z = gat_layer(a2, s2, enc2_b.reshape(1, -1), dec1_W, dec1_as,
                          dec1_ad, False, True)
    a4, s4, _ = gat_layer(a3, s3, dec1_b.reshape(1, -1), dec2_W, dec2_as,
                          dec2_ad, True, False)
    x_hat = _tc_combine(a4[0], s4, dec2_b.reshape(1, -1))
    q = _tc_q(z, cluster)
    return (z, x_hat, q)


# speedup vs baseline: 24.6649x; 1.3096x over previous
import functools

import jax
import jax.numpy as jnp
from jax import lax
from jax.experimental import pallas as pl
from jax.experimental.pallas import tpu as pltpu
from jax.experimental.pallas import tpu_sc as plsc

N = 10000
E_RAW = 320000
E_TOT = E_RAW + N
NW = 32
C = 96
CH = 108
PER_W = CH * C
E_PAD = PER_W * NW
BLK = 1000
GRID = N // BLK
RPT = N // 16


def _sc_gat(an, ad, h, src, dst, do_s):
    Dp = h.shape[1]
    mesh = plsc.VectorSubcoreMesh(core_axis_name="c", subcore_axis_name="s")

    @functools.partial(
        pl.kernel,
        out_type=(
            jax.ShapeDtypeStruct((2, N, Dp), jnp.float32),
            jax.ShapeDtypeStruct((2, N), jnp.float32),
        ),
        mesh=mesh,
        compiler_params=pltpu.CompilerParams(needs_layout_passes=False),
        scratch_types=(
            pltpu.VMEM((N,), jnp.float32),
            pltpu.VMEM((N,), jnp.float32),
            pltpu.VMEM((C,), jnp.int32),
            pltpu.VMEM((C,), jnp.int32),
            pltpu.VMEM((C,), jnp.int32),
            pltpu.VMEM((C,), jnp.int32),
            pltpu.VMEM((C,), jnp.float32),
            pltpu.VMEM((C, Dp), jnp.float32),
            pltpu.VMEM((C, Dp), jnp.float32),
            pltpu.VMEM((16, Dp), jnp.float32),
            pltpu.VMEM((800,), jnp.float32),
            pltpu.VMEM_SHARED((N, Dp), jnp.float32),
            pltpu.VMEM_SHARED((N,), jnp.float32),
            pltpu.SemaphoreType.DMA,
            pltpu.SemaphoreType.DMA,
        ),
    )
    def k(an_h, ad_h, h_h, src_h, dst_h, acc_o, s_o,
          an_v, ad_v, src_v0, dst_v0, src_v1, dst_v1, w_v,
          rows_v0, rows_v1, zbuf, zflat, acc_sh, s_sh, sem0, sem1):
        core = lax.axis_index("c")
        sid = lax.axis_index("s")
        i32 = jnp.int32
        iota = lax.iota(i32, 16)
        zeros16 = jnp.zeros((16,), jnp.float32)

        pltpu.sync_copy(an_h, an_v)
        pltpu.sync_copy(ad_h, ad_v)

        for r in range(16):
            for j in range(Dp // 16):
                zbuf[r, pl.ds(16 * j, 16)] = zeros16

        def za(kk, _):
            pltpu.sync_copy(zbuf, acc_sh.at[pl.ds(sid * 624 + kk * 16, 16)])
            return 0
        lax.fori_loop(0, 39, za, 0)

        @pl.when(sid == 15)
        def _ztail():
            pltpu.sync_copy(zbuf, acc_sh.at[pl.ds(9984, 16)])

        if do_s:
            def zf(i, _):
                zflat[pl.ds(i * 16, 16)] = zeros16
                return 0
            lax.fori_loop(0, 50, zf, 0)

            @pl.when(sid == 0)
            def _zs():
                for kk in range(12):
                    pltpu.sync_copy(zflat, s_sh.at[pl.ds(kk * 800, 800)])
                pltpu.sync_copy(zflat.at[pl.ds(0, 400)],
                                s_sh.at[pl.ds(9600, 400)])

        plsc.subcore_barrier()

        wid = core * 16 + sid
        base_w = wid * PER_W
        bufs = ((src_v0, dst_v0, rows_v0, sem0),
                (src_v1, dst_v1, rows_v1, sem1))

        pltpu.sync_copy(src_h.at[pl.ds(base_w, C)], src_v0)
        pltpu.sync_copy(dst_h.at[pl.ds(base_w, C)], dst_v0)
        pltpu.async_copy(h_h.at[src_v0], rows_v0, sem0)

        def body(t, _):
            for b in range(2):
                g = 2 * t + b
                sv, dv, rows, sem = bufs[b]
                svn, dvn, rowsn, semn = bufs[1 - b]

                @pl.when(g + 1 < CH)
                def _issue():
                    base_n = base_w + (g + 1) * C
                    pltpu.sync_copy(src_h.at[pl.ds(base_n, C)], svn)
                    pltpu.sync_copy(dst_h.at[pl.ds(base_n, C)], dvn)
                    pltpu.async_copy(h_h.at[svn], rowsn, semn)

                pltpu.make_async_copy(h_h.at[sv], rows, sem).wait()

                base = base_w + g * C
                for i in range(C // 16):
                    svv = sv[pl.ds(16 * i, 16)]
                    dvv = dv[pl.ds(16 * i, 16)]
                    a = plsc.load_gather(an_v, [svv])
                    b2 = plsc.load_gather(ad_v, [dvv])
                    e = a + b2
                    e = jnp.where(e >= 0.0, e, 0.2 * e)
                    w = jnp.exp(e)
                    eid = base + (16 * i) + iota
                    w = jnp.where(eid < E_TOT, w, 0.0)
                    w_v[pl.ds(16 * i, 16)] = w
                if do_s:
                    pltpu.sync_copy(w_v, s_sh.at[dv], add=True)

                def scale(r4, _):
                    for u in range(4):
                        r = r4 * 4 + u
                        wb = plsc.load_gather(w_v, [jnp.full((16,), r, i32)])
                        for j in range(Dp // 16):
                            rv = rows[r, pl.ds(16 * j, 16)]
                            rows[r, pl.ds(16 * j, 16)] = rv * wb
                    return 0
                lax.fori_loop(0, C // 4, scale, 0)
                pltpu.sync_copy(rows, acc_sh.at[dv], add=True)
            return 0
        lax.fori_loop(0, CH // 2, body, 0)

        plsc.subcore_barrier()
        pltpu.sync_copy(acc_sh.at[pl.ds(sid * 624, 624)],
                        acc_o.at[core, pl.ds(sid * 624, 624)])

        @pl.when(sid == 15)
        def _wtail():
            pltpu.sync_copy(acc_sh.at[pl.ds(9984, 16)],
                            acc_o.at[core, pl.ds(9984, 16)])

        if do_s:
            @pl.when(sid == 0)
            def _ws():
                pltpu.sync_copy(s_sh, s_o.at[core])

    return k(an, ad, h, src, dst)


def _tc_dense(parts_in, s2, bvec, W, asv, adv, relu, emit_t):
    Dout, Din = W.shape
    np_out = 2 if Dout > 128 else 1
    Dp_out = Dout // np_out
    n_in = len(parts_in)
    pro = s2 is not None

    out_shape = [jax.ShapeDtypeStruct((N, Dp_out), jnp.float32)
                 for _ in range(np_out)]
    out_shape += [jax.ShapeDtypeStruct((N, 1), jnp.float32)] * 2
    if emit_t:
        out_shape.append(jax.ShapeDtypeStruct((N, Din), jnp.float32))

    in_specs = []
    ops = [ ]
    if pro:
        Dpp = Din // n_in
        for p in parts_in:
            in_specs.append(pl.BlockSpec((2, BLK, Dpp), lambda i: (0, i, 0)))
            ops.append(p)
        in_specs.append(pl.BlockSpec((BLK, 2), lambda i: (i, 0)))
        ops.append(s2)
        in_specs.append(pl.BlockSpec((1, Din), lambda i: (0, 0)))
        ops.append(bvec)
    else:
        in_specs.append(pl.BlockSpec((BLK, Din), lambda i: (i, 0)))
        ops.append(parts_in[0])
    in_specs.append(pl.BlockSpec((Dout, Din), lambda i: (0, 0)))
    ops.append(W)
    in_specs.append(pl.BlockSpec((1, Dout), lambda i: (0, 0)))
    ops.append(asv)
    in_specs.append(pl.BlockSpec((1, Dout), lambda i: (0, 0)))
    ops.append(adv)

    out_specs = [pl.BlockSpec((BLK, Dp_out), lambda i: (i, 0))
                 for _ in range(np_out)]
    out_specs += [pl.BlockSpec((BLK, 1), lambda i: (i, 0))] * 2
    if emit_t:
        out_specs.append(pl.BlockSpec((BLK, Din), lambda i: (i, 0)))

    def body(*refs):
        pos = 0
        if pro:
            prefs = refs[:n_in]
            s_ref = refs[n_in]
            b_ref = refs[n_in + 1]
            pos = n_in + 2
        else:
            prefs = (refs[0],)
            pos = 1
        W_ref, as_ref, ad_ref = refs[pos], refs[pos + 1], refs[pos + 2]
        out_refs = refs[pos + 3:]
        if pro:
            cols = [pr[0] + pr[1] for pr in prefs]
            t = cols[0] if n_in == 1 else jnp.concatenate(cols, axis=1)
            ssum = s_ref[:, 0:1] + s_ref[:, 1:2]
            t = t / ssum + b_ref[...]
            if relu:
                t = jnp.maximum(t, 0.0)
        else:
            t = prefs[0][...]
        h = lax.dot_general(t, W_ref[...], (((1,), (1,)), ((), ())),
                            preferred_element_type=jnp.float32)
        an = jnp.sum(h * as_ref[...], axis=1, keepdims=True)
        ad = jnp.sum(h * ad_ref[...], axis=1, keepdims=True)
        o = 0
        for p in range(np_out):
            out_refs[o][...] = h[:, p * Dp_out:(p + 1) * Dp_out]
            o += 1
        out_refs[o][...] = an
        o += 1
        out_refs[o][...] = ad
        o += 1
        if emit_t:
            out_refs[o][...] = t

    return pl.pallas_call(
        body, grid=(GRID,), in_specs=in_specs, out_specs=out_specs,
        out_shape=out_shape)(*ops)


def _tc_combine(acc, s2, bvec):
    Dp = acc.shape[2]

    def body(a_ref, s_ref, b_ref, o_ref):
        ssum = s_ref[:, 0:1] + s_ref[:, 1:2]
        o_ref[...] = (a_ref[0] + a_ref[1]) / ssum + b_ref[...]

    return pl.pallas_call(
        body, grid=(GRID,),
        in_specs=[pl.BlockSpec((2, BLK, Dp), lambda i: (0, i, 0)),
                  pl.BlockSpec((BLK, 2), lambda i: (i, 0)),
                  pl.BlockSpec((1, Dp), lambda i: (0, 0))],
        out_specs=pl.BlockSpec((BLK, Dp), lambda i: (i, 0)),
        out_shape=jax.ShapeDtypeStruct((N, Dp), jnp.float32),
    )(acc, s2, bvec)


def _tc_q(z, cluster):
    K, Zd = cluster.shape

    def body(z_ref, c_ref, o_ref):
        zb = z_ref[...]
        cb = c_ref[...]
        zz = jnp.sum(zb * zb, axis=1, keepdims=True)
        cc = jnp.sum(cb * cb, axis=1).reshape(1, K)
        d2 = zz + cc - 2.0 * lax.dot_general(
            zb, cb, (((1,), (1,)), ((), ())),
            preferred_element_type=jnp.float32)
        q = jnp.sqrt(1.0 / (1.0 + d2))
        o_ref[...] = q / jnp.sum(q, axis=1, keepdims=True)

    return pl.pallas_call(
        body, grid=(GRID,),
        in_specs=[pl.BlockSpec((BLK, Zd), lambda i: (i, 0)),
                  pl.BlockSpec((K, Zd), lambda i: (0, 0))],
        out_specs=pl.BlockSpec((BLK, K), lambda i: (i, 0)),
        out_shape=jax.ShapeDtypeStruct((N, K), jnp.float32),
    )(z, cluster)


def kernel(x, edge_index, enc1_W, enc1_as, enc1_ad, enc1_b,
           enc2_W, enc2_as, enc2_ad, enc2_b,
           dec1_W, dec1_as, dec1_ad, dec1_b,
           dec2_W, dec2_as, dec2_ad, dec2_b, cluster):
    ei = edge_index.astype(jnp.int32)
    loops = jnp.arange(N, dtype=jnp.int32)
    n_pad = E_PAD - E_TOT
    pad_idx = (jnp.arange(n_pad, dtype=jnp.int32) * 97) % N
    src = jnp.concatenate([ei[0], loops, pad_idx])
    dst = jnp.concatenate([ei[1], loops, pad_idx])

    def gat_layer(parts_in, s_prev, b_prev, W, a_s, a_d, relu, emit_t):
        outs = _tc_dense(parts_in, s_prev, b_prev, W,
                         a_s.reshape(1, -1), a_d.reshape(1, -1), relu, emit_t)
        np_out = 2 if W.shape[0] > 128 else 1
        hparts = outs[:np_out]
        an = outs[np_out].reshape(N)
        ad = outs[np_out + 1].reshape(N)
        t = outs[np_out + 2] if emit_t else None
        accs = []
        s_pair = None
        for p, hp in enumerate(hparts):
            d = hp.shape[1]
            if d < 128:
                hp = jnp.pad(hp, ((0, 0), (0, 128 - d)))
            acc, s_parts = _sc_gat(an, ad, hp, src, dst, p == 0)
            if d < 128:
                acc = acc[:, :, :d]
            accs.append(acc)
            if p == 0:
                s_pair = s_parts.T
        return accs, s_pair, t

    a1, s1, _ = gat_layer([x], None, None, enc1_W, enc1_as, enc1_ad,
                          False, False)
    a2, s2, _ = gat_layer(a1, s1, enc1_b.reshape(1, -1), enc2_W, enc2_as,
                          enc2_ad, True, False)
    a3, s3, z = gat_layer(a2, s2, enc2_b.reshape(1, -1), dec1_W, dec1_as,
                          dec1_ad, False, True)
    a4, s4, _ = gat_layer(a3, s3, dec1_b.reshape(1, -1), dec2_W, dec2_as,
                          dec2_ad, True, False)
    x_hat = _tc_combine(a4[0], s4, dec2_b.reshape(1, -1))
    q = _tc_q(z, cluster)
    return (z, x_hat, q)
